# Initial kernel scaffold; baseline (speedup 1.0000x reference)
import jax
import jax.numpy as jnp
from jax.experimental import pallas as pl


def kernel(x):
    raise NotImplementedError("write your pallas kernel here")


# speedup vs baseline: 24.0774x; 24.0774x over previous
import functools

import jax
import jax.numpy as jnp
from jax.experimental import pallas as pl
from jax.experimental.pallas import tpu as pltpu

RATIO = 0.1


def _kwta_body(k, x_ref, o_ref):
    xb = x_ref[...]
    u = pltpu.bitcast(xb, jnp.uint32)
    neg = (u >> 31).astype(jnp.uint32)
    key = pltpu.bitcast(u ^ (neg * jnp.uint32(0x7FFFFFFF)), jnp.int32)

    p = xb.shape[0]
    kvec = jnp.full((p,), k, dtype=jnp.int32)

    def count_ge(t):
        m = key >= t[:, None, None]
        return jnp.sum(m.astype(jnp.int32), axis=(1, 2))

    c = count_ge(jnp.zeros((p,), jnp.int32))
    lo = jnp.where(c >= kvec, jnp.int32(0), jnp.int32(-2147483648))
    for b in range(30, -1, -1):
        t = lo | jnp.int32(1 << b)
        c = count_ge(t)
        lo = jnp.where(c >= kvec, t, lo)

    mask = key >= lo[:, None, None]
    o_ref[...] = jnp.where(mask, xb, jnp.float32(0.0))


def _kwta_planes(xp, k, p_group):
    n, s, l = xp.shape
    grid = n // p_group
    return pl.pallas_call(
        functools.partial(_kwta_body, k),
        grid=(grid,),
        in_specs=[pl.BlockSpec((p_group, s, l), lambda i: (i, 0, 0))],
        out_specs=pl.BlockSpec((p_group, s, l), lambda i: (i, 0, 0)),
        out_shape=jax.ShapeDtypeStruct((n, s, l), jnp.float32),
        compiler_params=pltpu.CompilerParams(
            dimension_semantics=("arbitrary",),
        ),
    )(xp)


def kernel(x):
    b, c, h, w = x.shape
    size = h * w
    k = int(RATIO * size)
    n = b * c
    lanes = 128
    assert size % lanes == 0
    s = size // lanes
    xp = x.reshape(n, s, lanes)
    p_group = 8
    while n % p_group:
        p_group //= 2
    out = _kwta_planes(xp, k, p_group)
    return out.reshape(b, c, h, w)


# speedup vs baseline: 27.5516x; 1.1443x over previous
import functools

import jax
import jax.numpy as jnp
from jax.experimental import pallas as pl
from jax.experimental.pallas import tpu as pltpu

RATIO = 0.1


def _kwta_body(k, x_ref, o_ref):
    xb = x_ref[...]
    u = pltpu.bitcast(xb, jnp.uint32)
    neg = (u >> 31).astype(jnp.uint32)
    key = pltpu.bitcast(u ^ (neg * jnp.uint32(0x7FFFFFFF)), jnp.int32)

    p = xb.shape[0]
    kvec = jnp.full((p,), k, dtype=jnp.int32)

    def count_ge(t):
        m = key >= t[:, None, None]
        return jnp.sum(m.astype(jnp.int32), axis=(1, 2))

    c = count_ge(jnp.zeros((p,), jnp.int32))
    lo = jnp.where(c >= kvec, jnp.int32(0), jnp.int32(-2147483648))
    for b in range(30, -1, -1):
        t = lo | jnp.int32(1 << b)
        c = count_ge(t)
        lo = jnp.where(c >= kvec, t, lo)

    mask = key >= lo[:, None, None]
    o_ref[...] = jnp.where(mask, xb, jnp.float32(0.0))


def _kwta_planes(xp, k, p_group):
    n, s, l = xp.shape
    grid = n // p_group
    return pl.pallas_call(
        functools.partial(_kwta_body, k),
        grid=(grid,),
        in_specs=[pl.BlockSpec((p_group, s, l), lambda i: (i, 0, 0))],
        out_specs=pl.BlockSpec((p_group, s, l), lambda i: (i, 0, 0)),
        out_shape=jax.ShapeDtypeStruct((n, s, l), jnp.float32),
        compiler_params=pltpu.CompilerParams(
            dimension_semantics=("arbitrary",),
        ),
    )(xp)


def kernel(x):
    b, c, h, w = x.shape
    size = h * w
    k = int(RATIO * size)
    n = b * c
    lanes = 128
    assert size % lanes == 0
    s = size // lanes
    xp = x.reshape(n, s, lanes)
    p_group = 16
    while n % p_group:
        p_group //= 2
    out = _kwta_planes(xp, k, p_group)
    return out.reshape(b, c, h, w)


# speedup vs baseline: 29.5872x; 1.0739x over previous
import functools

import jax
import jax.numpy as jnp
from jax.experimental import pallas as pl
from jax.experimental.pallas import tpu as pltpu

RATIO = 0.1


def _kwta_body(k, x_ref, o_ref):
    xb = x_ref[...]
    u = pltpu.bitcast(xb, jnp.uint32)
    neg = (u >> 31).astype(jnp.uint32)
    key = pltpu.bitcast(u ^ (neg * jnp.uint32(0x7FFFFFFF)), jnp.int32)

    p = xb.shape[0]
    kvec = jnp.full((p,), k, dtype=jnp.int32)

    def count_ge(t):
        m = key >= t[:, None, None]
        return jnp.sum(m.astype(jnp.int32), axis=(1, 2))

    c = count_ge(jnp.zeros((p,), jnp.int32))
    lo = jnp.where(c >= kvec, jnp.int32(0), jnp.int32(-2147483648))
    for b in range(30, -1, -1):
        t = lo | jnp.int32(1 << b)
        c = count_ge(t)
        lo = jnp.where(c >= kvec, t, lo)

    mask = key >= lo[:, None, None]
    o_ref[...] = jnp.where(mask, xb, jnp.float32(0.0))


def _kwta_planes(xp, k, p_group):
    n, s, l = xp.shape
    grid = n // p_group
    return pl.pallas_call(
        functools.partial(_kwta_body, k),
        grid=(grid,),
        in_specs=[pl.BlockSpec((p_group, s, l), lambda i: (i, 0, 0))],
        out_specs=pl.BlockSpec((p_group, s, l), lambda i: (i, 0, 0)),
        out_shape=jax.ShapeDtypeStruct((n, s, l), jnp.float32),
        compiler_params=pltpu.CompilerParams(
            dimension_semantics=("arbitrary",),
        ),
    )(xp)


def kernel(x):
    b, c, h, w = x.shape
    size = h * w
    k = int(RATIO * size)
    n = b * c
    lanes = 128
    assert size % lanes == 0
    s = size // lanes
    xp = x.reshape(n, s, lanes)
    p_group = 32
    while n % p_group:
        p_group //= 2
    out = _kwta_planes(xp, k, p_group)
    return out.reshape(b, c, h, w)
